# baseline (speedup 1.0000x reference)
import functools

import jax
import jax.numpy as jnp
from jax import lax
from jax.experimental import pallas as pl
from jax.experimental.pallas import tpu as pltpu
from jax.experimental.pallas import tpu_sc as plsc

BINSIZE = 200
FS_WIDTH = 1024.0
N_ROI = 64
N_CLUSTERS = 8
BINWIDTH = 500
TABLE_WORDS = N_ROI * N_CLUSTERS * BINWIDTH
N_FRAG = 1_000_000
NW = 32
CHUNK = 1008
GROUPS = CHUNK // 16
NCHUNK = 31
PER_TILE = CHUNK * NCHUNK
TAIL_BASE = PER_TILE * NW
TAIL = N_FRAG - TAIL_BASE
LABEL_WORDS = 1250

_LOG2_C = (-2.7868055642987652, 5.046852935527453, -3.4924660425540925,
           1.5938845482669501, -0.40486230941537504, 0.04342836333154978)
_LN2 = 0.6931471805599453


def _table_body(base_ref, delta_ref, out_ref):
    u = base_ref[0] + delta_ref[0]
    m = jnp.max(u, axis=-1, keepdims=True)
    lse = m + jnp.log(jnp.sum(jnp.exp(u - m), axis=-1, keepdims=True))
    out_ref[0] = u - lse


def _build_table(regions_oi, baseline_weight, delta_logit_weight):
    base_oi = baseline_weight[regions_oi].reshape(N_ROI, 1, BINWIDTH)
    delta_oi = delta_logit_weight[regions_oi]
    table = pl.pallas_call(
        _table_body,
        grid=(N_ROI,),
        in_specs=[
            pl.BlockSpec((1, 1, BINWIDTH), lambda i: (i, 0, 0)),
            pl.BlockSpec((1, N_CLUSTERS, BINWIDTH), lambda i: (i, 0, 0)),
        ],
        out_specs=pl.BlockSpec((1, N_CLUSTERS, BINWIDTH),
                               lambda i: (i, 0, 0)),
        out_shape=jax.ShapeDtypeStruct((N_ROI, N_CLUSTERS, BINWIDTH),
                                       jnp.float32),
    )(base_oi, delta_oi)
    return table.reshape(-1)


def _emit_idx_lad(jb, in1_b, in2_b, idx_b, out_b, labels_v, g_v, lpi, cout,
                  iota16):
    sl = pl.ds(jb, 16)
    w1 = in1_b[sl]
    w2 = in2_b[sl]
    coord = w1 & 0x1FFFF
    fs = lax.shift_right_logical(w1, 17)
    cell = w2 & 0x3FFF
    region = lax.shift_right_logical(w2, 14)
    lw = plsc.load_gather(labels_v, [lax.shift_right_logical(cell, 3)])
    cl = lax.shift_right_logical(lw, lax.shift_left(cell & 7, 2)) & 7
    bin_ = lax.div(coord, BINSIZE)
    idx_b[sl] = (region * N_CLUSTERS + cl) * BINWIDTH + bin_
    fsf = fs.astype(jnp.float32)
    xb = jnp.clip(fsf * (1.0 / FS_WIDTH), 0.0, 1.0) * 4.0
    b = jnp.minimum(xb.astype(jnp.int32), 3)
    alpha = xb - b.astype(jnp.float32)
    gb = plsc.load_gather(g_v, [b])
    gb1 = plsc.load_gather(g_v, [b + 1])
    dens = gb + alpha * (gb1 - gb) + 1e-12
    ib = plsc.bitcast(dens, jnp.int32)
    e = (lax.shift_right_logical(ib, 23) & 0xFF) - 127
    m = plsc.bitcast((ib & 0x7FFFFF) | 0x3F800000, jnp.float32)
    p = jnp.float32(_LOG2_C[5])
    for k in (4, 3, 2, 1, 0):
        p = p * m + jnp.float32(_LOG2_C[k])
    ln = (e.astype(jnp.float32) + p) * _LN2
    lad = jnp.where(fsf > FS_WIDTH, cout, ln + lpi)
    plsc.store_scatter(out_b, [jb + iota16, iota16 * 0 + 1], lad)


def _sc_body(table_hbm, labels_hbm, w1_hbm, w2_hbm, consts_hbm, out_hbm,
             labels_v, g_v, in1_v0, in1_v1, in2_v0, in2_v1,
             idx_v, gath_v, out_v0, out_v1,
             s10, s11, s20, s21, so0, so1, sg):
    in1_bufs = (in1_v0, in1_v1)
    in2_bufs = (in2_v0, in2_v1)
    out_bufs = (out_v0, out_v1)
    in1_sems = (s10, s11)
    in2_sems = (s20, s21)
    out_sems = (so0, so1)
    wid = lax.axis_index("s") * 2 + lax.axis_index("c")
    tile_base = wid * PER_TILE
    pltpu.sync_copy(labels_hbm, labels_v)
    pltpu.sync_copy(consts_hbm, g_v)
    iota16 = lax.iota(jnp.int32, 16)
    c5 = iota16 * 0 + 5
    lpi = plsc.load_gather(g_v, [c5])
    cout = plsc.load_gather(g_v, [c5 + 1])

    def in1_copy(g, b):
        base = tile_base + g * CHUNK
        return pltpu.make_async_copy(
            w1_hbm.at[pl.ds(base, CHUNK)], in1_bufs[b], in1_sems[b])

    def in2_copy(g, b):
        base = tile_base + g * CHUNK
        return pltpu.make_async_copy(
            w2_hbm.at[pl.ds(base, CHUNK)], in2_bufs[b], in2_sems[b])

    def in_start(g, b):
        in1_copy(g, b).start()
        in2_copy(g, b).start()

    def in_wait(g, b):
        in1_copy(g, b).wait()
        in2_copy(g, b).wait()

    def out_copy(g, b):
        base = tile_base + g * CHUNK
        return pltpu.make_async_copy(
            out_bufs[b], out_hbm.at[pl.ds(base, CHUNK)], out_sems[b])

    def scatter_lp0(out_b, jb, lp0):
        plsc.store_scatter(out_b, [jb + iota16, iota16 * 0], lp0)

    def compute(b):
        def blk1(t, carry):
            for jj in range(9):
                _emit_idx_lad(t * 144 + jj * 16, in1_bufs[b], in2_bufs[b],
                              idx_v, out_bufs[b], labels_v, g_v, lpi, cout,
                              iota16)
            return carry

        lax.fori_loop(0, GROUPS // 9, blk1, 0)
        pltpu.async_copy(table_hbm.at[idx_v], gath_v, sg).wait()

        def blk2(t, carry):
            for jj in range(9):
                jb = t * 144 + jj * 16
                scatter_lp0(out_bufs[b], jb, gath_v[pl.ds(jb, 16)])
            return carry

        lax.fori_loop(0, GROUPS // 9, blk2, 0)

    in_start(0, 0)
    in_start(1, 1)
    in_wait(0, 0)
    compute(0)
    out_copy(0, 0).start()

    def pair_body(i, carry):
        for off, b in ((0, 1), (1, 0)):
            g = 1 + 2 * i + off

            @pl.when(g + 1 < NCHUNK)
            def _():
                in_start(g + 1, 1 - b)

            in_wait(g, b)

            @pl.when(g >= 2)
            def _():
                out_copy(g - 2, b).wait()

            compute(b)
            out_copy(g, b).start()
        return carry

    lax.fori_loop(0, (NCHUNK - 1) // 2, pair_body, 0)
    out_copy(NCHUNK - 2, 1).wait()
    out_copy(NCHUNK - 1, 0).wait()

    @pl.when(wid == 0)
    def _tail():
        pltpu.sync_copy(w1_hbm.at[pl.ds(TAIL_BASE, TAIL)],
                        in1_v0.at[pl.ds(0, TAIL)])
        pltpu.sync_copy(w2_hbm.at[pl.ds(TAIL_BASE, TAIL)],
                        in2_v0.at[pl.ds(0, TAIL)])
        for j in range(TAIL // 16):
            _emit_idx_lad(j * 16, in1_v0, in2_v0, idx_v, out_v0,
                          labels_v, g_v, lpi, cout, iota16)
        pltpu.async_copy(table_hbm.at[idx_v.at[pl.ds(0, TAIL)]],
                         gath_v.at[pl.ds(0, TAIL)], sg).wait()
        for j in range(TAIL // 16):
            scatter_lp0(out_v0, j * 16, gath_v[pl.ds(j * 16, 16)])
        pltpu.sync_copy(out_v0.at[pl.ds(0, TAIL)],
                        out_hbm.at[pl.ds(TAIL_BASE, TAIL)])


@functools.cache
def _sc_call():
    return pl.kernel(
        _sc_body,
        out_type=jax.ShapeDtypeStruct((N_FRAG, 2), jnp.float32),
        mesh=plsc.VectorSubcoreMesh(core_axis_name="c", subcore_axis_name="s",
                                    num_cores=2, num_subcores=16),
        compiler_params=pltpu.CompilerParams(needs_layout_passes=False,
                                             use_tc_tiling_on_sc=False),
        scratch_types=[
            pltpu.VMEM((LABEL_WORDS,), jnp.int32),
            pltpu.VMEM((128,), jnp.float32),
            pltpu.VMEM((CHUNK,), jnp.int32),
            pltpu.VMEM((CHUNK,), jnp.int32),
            pltpu.VMEM((CHUNK,), jnp.int32),
            pltpu.VMEM((CHUNK,), jnp.int32),
            pltpu.VMEM((CHUNK,), jnp.int32),
            pltpu.VMEM((CHUNK,), jnp.float32),
            pltpu.VMEM((CHUNK, 2), jnp.float32),
            pltpu.VMEM((CHUNK, 2), jnp.float32),
            pltpu.SemaphoreType.DMA,
            pltpu.SemaphoreType.DMA,
            pltpu.SemaphoreType.DMA,
            pltpu.SemaphoreType.DMA,
            pltpu.SemaphoreType.DMA,
            pltpu.SemaphoreType.DMA,
            pltpu.SemaphoreType.DMA,
        ],
    )


def kernel(coord_left, fragment_size, regions_oi, local_region_ix,
           local_cell_ix, labels, baseline_weight, delta_logit_weight,
           spline_unnorm_heights, logprob_inside, spline_widths,
           spline_bin_locations):
    table = _build_table(regions_oi, baseline_weight, delta_logit_weight)
    lw = labels.astype(jnp.int32).reshape(LABEL_WORDS, 8)
    shifts = (jnp.arange(8, dtype=jnp.int32) * 4)[None, :]
    labels_words = jnp.sum(lw << shifts, axis=1, dtype=jnp.int32)
    w1 = coord_left.astype(jnp.int32) | (fragment_size.astype(jnp.int32) << 17)
    w2 = local_cell_ix.astype(jnp.int32) | (local_region_ix.astype(jnp.int32) << 14)
    h = jnp.exp(spline_unnorm_heights)
    norm = jnp.sum((h[:-1] + h[1:]) * 0.5 * spline_widths)
    g = (h / norm).astype(jnp.float32)
    lpi = logprob_inside.astype(jnp.float32)
    cout = jnp.log(1.0 - jnp.exp(lpi)) + lpi
    consts = jnp.concatenate(
        [g, lpi[None], cout[None], jnp.zeros((121,), jnp.float32)])
    return _sc_call()(table, labels_words, w1, w2, consts)


# speedup vs baseline: 1.6144x; 1.6144x over previous
import functools

import jax
import jax.numpy as jnp
from jax import lax
from jax.experimental import pallas as pl
from jax.experimental.pallas import tpu as pltpu
from jax.experimental.pallas import tpu_sc as plsc

BINSIZE = 200
FS_WIDTH = 1024.0
N_ROI = 64
N_CLUSTERS = 8
BINWIDTH = 500
TABLE_WORDS = N_ROI * N_CLUSTERS * BINWIDTH
N_FRAG = 1_000_000
NW = 32
CHUNK = 1008
GROUPS = CHUNK // 16
NCHUNK = 31
PER_TILE = CHUNK * NCHUNK
TAIL_BASE = PER_TILE * NW
TAIL = N_FRAG - TAIL_BASE
LABEL_WORDS = 1250

_LOG2_C = (-2.7868055642987652, 5.046852935527453, -3.4924660425540925,
           1.5938845482669501, -0.40486230941537504, 0.04342836333154978)
_LN2 = 0.6931471805599453


def _table_body(roi_ref, base_ref, delta_ref, out_ref):
    u = base_ref[0] + delta_ref[0]
    m = jnp.max(u, axis=-1, keepdims=True)
    lse = m + jnp.log(jnp.sum(jnp.exp(u - m), axis=-1, keepdims=True))
    out_ref[0] = u - lse


def _build_table(regions_oi, baseline_weight, delta_logit_weight):
    n_regions = baseline_weight.shape[0]
    table = pl.pallas_call(
        _table_body,
        grid_spec=pltpu.PrefetchScalarGridSpec(
            num_scalar_prefetch=1,
            grid=(N_ROI,),
            in_specs=[
                pl.BlockSpec((1, 1, BINWIDTH), lambda i, roi: (roi[i], 0, 0)),
                pl.BlockSpec((1, N_CLUSTERS, BINWIDTH),
                             lambda i, roi: (roi[i], 0, 0)),
            ],
            out_specs=pl.BlockSpec((1, N_CLUSTERS, BINWIDTH),
                                   lambda i, roi: (i, 0, 0)),
        ),
        out_shape=jax.ShapeDtypeStruct((N_ROI, N_CLUSTERS, BINWIDTH),
                                       jnp.float32),
    )(regions_oi, baseline_weight.reshape(n_regions, 1, BINWIDTH),
      delta_logit_weight)
    return table.reshape(-1)


def _emit_idx_lad(jb, in1_b, in2_b, idx_b, out_b, labels_v, g_v, lpi, cout,
                  iota16):
    sl = pl.ds(jb, 16)
    w1 = in1_b[sl]
    w2 = in2_b[sl]
    coord = w1 & 0x1FFFF
    fs = lax.shift_right_logical(w1, 17)
    cell = w2 & 0x3FFF
    region = lax.shift_right_logical(w2, 14)
    lw = plsc.load_gather(labels_v, [lax.shift_right_logical(cell, 3)])
    cl = lax.shift_right_logical(lw, lax.shift_left(cell & 7, 2)) & 7
    bin_ = lax.div(coord, BINSIZE)
    idx_b[sl] = (region * N_CLUSTERS + cl) * BINWIDTH + bin_
    fsf = fs.astype(jnp.float32)
    xb = jnp.clip(fsf * (1.0 / FS_WIDTH), 0.0, 1.0) * 4.0
    b = jnp.minimum(xb.astype(jnp.int32), 3)
    alpha = xb - b.astype(jnp.float32)
    gb = plsc.load_gather(g_v, [b])
    gb1 = plsc.load_gather(g_v, [b + 1])
    dens = gb + alpha * (gb1 - gb) + 1e-12
    ib = plsc.bitcast(dens, jnp.int32)
    e = (lax.shift_right_logical(ib, 23) & 0xFF) - 127
    m = plsc.bitcast((ib & 0x7FFFFF) | 0x3F800000, jnp.float32)
    p = jnp.float32(_LOG2_C[5])
    for k in (4, 3, 2, 1, 0):
        p = p * m + jnp.float32(_LOG2_C[k])
    ln = (e.astype(jnp.float32) + p) * _LN2
    lad = jnp.where(fsf > FS_WIDTH, cout, ln + lpi)
    plsc.store_scatter(out_b, [jb + iota16, iota16 * 0 + 1], lad)


def _sc_body(table_hbm, labels_hbm, w1_hbm, w2_hbm, consts_hbm, out_hbm,
             labels_v, g_v, in1_v0, in1_v1, in2_v0, in2_v1,
             idx_v, gath_v, out_v0, out_v1,
             s10, s11, s20, s21, so0, so1, sg):
    in1_bufs = (in1_v0, in1_v1)
    in2_bufs = (in2_v0, in2_v1)
    out_bufs = (out_v0, out_v1)
    in1_sems = (s10, s11)
    in2_sems = (s20, s21)
    out_sems = (so0, so1)
    wid = lax.axis_index("s") * 2 + lax.axis_index("c")
    tile_base = wid * PER_TILE
    pltpu.sync_copy(labels_hbm, labels_v)
    pltpu.sync_copy(consts_hbm, g_v)
    iota16 = lax.iota(jnp.int32, 16)
    c5 = iota16 * 0 + 5
    lpi = plsc.load_gather(g_v, [c5])
    cout = plsc.load_gather(g_v, [c5 + 1])

    def in1_copy(g, b):
        base = tile_base + g * CHUNK
        return pltpu.make_async_copy(
            w1_hbm.at[pl.ds(base, CHUNK)], in1_bufs[b], in1_sems[b])

    def in2_copy(g, b):
        base = tile_base + g * CHUNK
        return pltpu.make_async_copy(
            w2_hbm.at[pl.ds(base, CHUNK)], in2_bufs[b], in2_sems[b])

    def in_start(g, b):
        in1_copy(g, b).start()
        in2_copy(g, b).start()

    def in_wait(g, b):
        in1_copy(g, b).wait()
        in2_copy(g, b).wait()

    def out_copy(g, b):
        base = tile_base + g * CHUNK
        return pltpu.make_async_copy(
            out_bufs[b], out_hbm.at[pl.ds(base, CHUNK)], out_sems[b])

    def scatter_lp0(out_b, jb, lp0):
        plsc.store_scatter(out_b, [jb + iota16, iota16 * 0], lp0)

    def compute(b):
        def blk1(t, carry):
            for jj in range(9):
                _emit_idx_lad(t * 144 + jj * 16, in1_bufs[b], in2_bufs[b],
                              idx_v, out_bufs[b], labels_v, g_v, lpi, cout,
                              iota16)
            return carry

        lax.fori_loop(0, GROUPS // 9, blk1, 0)
        pltpu.async_copy(table_hbm.at[idx_v], gath_v, sg).wait()

        def blk2(t, carry):
            for jj in range(9):
                jb = t * 144 + jj * 16
                scatter_lp0(out_bufs[b], jb, gath_v[pl.ds(jb, 16)])
            return carry

        lax.fori_loop(0, GROUPS // 9, blk2, 0)

    in_start(0, 0)
    in_start(1, 1)
    in_wait(0, 0)
    compute(0)
    out_copy(0, 0).start()

    def pair_body(i, carry):
        for off, b in ((0, 1), (1, 0)):
            g = 1 + 2 * i + off

            @pl.when(g + 1 < NCHUNK)
            def _():
                in_start(g + 1, 1 - b)

            in_wait(g, b)

            @pl.when(g >= 2)
            def _():
                out_copy(g - 2, b).wait()

            compute(b)
            out_copy(g, b).start()
        return carry

    lax.fori_loop(0, (NCHUNK - 1) // 2, pair_body, 0)
    out_copy(NCHUNK - 2, 1).wait()
    out_copy(NCHUNK - 1, 0).wait()

    @pl.when(wid == 0)
    def _tail():
        pltpu.sync_copy(w1_hbm.at[pl.ds(TAIL_BASE, TAIL)],
                        in1_v0.at[pl.ds(0, TAIL)])
        pltpu.sync_copy(w2_hbm.at[pl.ds(TAIL_BASE, TAIL)],
                        in2_v0.at[pl.ds(0, TAIL)])
        for j in range(TAIL // 16):
            _emit_idx_lad(j * 16, in1_v0, in2_v0, idx_v, out_v0,
                          labels_v, g_v, lpi, cout, iota16)
        pltpu.async_copy(table_hbm.at[idx_v.at[pl.ds(0, TAIL)]],
                         gath_v.at[pl.ds(0, TAIL)], sg).wait()
        for j in range(TAIL // 16):
            scatter_lp0(out_v0, j * 16, gath_v[pl.ds(j * 16, 16)])
        pltpu.sync_copy(out_v0.at[pl.ds(0, TAIL)],
                        out_hbm.at[pl.ds(TAIL_BASE, TAIL)])


@functools.cache
def _sc_call():
    return pl.kernel(
        _sc_body,
        out_type=jax.ShapeDtypeStruct((N_FRAG, 2), jnp.float32),
        mesh=plsc.VectorSubcoreMesh(core_axis_name="c", subcore_axis_name="s",
                                    num_cores=2, num_subcores=16),
        compiler_params=pltpu.CompilerParams(needs_layout_passes=False,
                                             use_tc_tiling_on_sc=False),
        scratch_types=[
            pltpu.VMEM((LABEL_WORDS,), jnp.int32),
            pltpu.VMEM((128,), jnp.float32),
            pltpu.VMEM((CHUNK,), jnp.int32),
            pltpu.VMEM((CHUNK,), jnp.int32),
            pltpu.VMEM((CHUNK,), jnp.int32),
            pltpu.VMEM((CHUNK,), jnp.int32),
            pltpu.VMEM((CHUNK,), jnp.int32),
            pltpu.VMEM((CHUNK,), jnp.float32),
            pltpu.VMEM((CHUNK, 2), jnp.float32),
            pltpu.VMEM((CHUNK, 2), jnp.float32),
            pltpu.SemaphoreType.DMA,
            pltpu.SemaphoreType.DMA,
            pltpu.SemaphoreType.DMA,
            pltpu.SemaphoreType.DMA,
            pltpu.SemaphoreType.DMA,
            pltpu.SemaphoreType.DMA,
            pltpu.SemaphoreType.DMA,
        ],
    )


def kernel(coord_left, fragment_size, regions_oi, local_region_ix,
           local_cell_ix, labels, baseline_weight, delta_logit_weight,
           spline_unnorm_heights, logprob_inside, spline_widths,
           spline_bin_locations):
    table = _build_table(regions_oi, baseline_weight, delta_logit_weight)
    lw = labels.astype(jnp.int32).reshape(LABEL_WORDS, 8)
    shifts = (jnp.arange(8, dtype=jnp.int32) * 4)[None, :]
    labels_words = jnp.sum(lw << shifts, axis=1, dtype=jnp.int32)
    w1 = coord_left.astype(jnp.int32) | (fragment_size.astype(jnp.int32) << 17)
    w2 = local_cell_ix.astype(jnp.int32) | (local_region_ix.astype(jnp.int32) << 14)
    h = jnp.exp(spline_unnorm_heights)
    norm = jnp.sum((h[:-1] + h[1:]) * 0.5 * spline_widths)
    g = (h / norm).astype(jnp.float32)
    lpi = logprob_inside.astype(jnp.float32)
    cout = jnp.log(1.0 - jnp.exp(lpi)) + lpi
    consts = jnp.concatenate(
        [g, lpi[None], cout[None], jnp.zeros((121,), jnp.float32)])
    return _sc_call()(table, labels_words, w1, w2, consts)


# speedup vs baseline: 1.6994x; 1.0527x over previous
import functools

import jax
import jax.numpy as jnp
from jax import lax
from jax.experimental import pallas as pl
from jax.experimental.pallas import tpu as pltpu
from jax.experimental.pallas import tpu_sc as plsc

BINSIZE = 200
FS_WIDTH = 1024.0
N_ROI = 64
N_CLUSTERS = 8
BINWIDTH = 500
TABLE_WORDS = N_ROI * N_CLUSTERS * BINWIDTH
N_FRAG = 1_000_000
NW = 32
CHUNK = 1008
GROUPS = CHUNK // 16
NCHUNK = 31
PER_TILE = CHUNK * NCHUNK
TAIL_BASE = PER_TILE * NW
TAIL = N_FRAG - TAIL_BASE
LABEL_WORDS = 1250

_LOG2_C = (-2.7868055642987652, 5.046852935527453, -3.4924660425540925,
           1.5938845482669501, -0.40486230941537504, 0.04342836333154978)
_LN2 = 0.6931471805599453


def _table_body(roi_ref, base_ref, delta_ref, out_ref):
    u = base_ref[0] + delta_ref[0]
    m = jnp.max(u, axis=-1, keepdims=True)
    lse = m + jnp.log(jnp.sum(jnp.exp(u - m), axis=-1, keepdims=True))
    out_ref[0] = u - lse


def _build_table(regions_oi, baseline_weight, delta_logit_weight):
    n_regions = baseline_weight.shape[0]
    table = pl.pallas_call(
        _table_body,
        grid_spec=pltpu.PrefetchScalarGridSpec(
            num_scalar_prefetch=1,
            grid=(N_ROI,),
            in_specs=[
                pl.BlockSpec((1, 1, BINWIDTH), lambda i, roi: (roi[i], 0, 0)),
                pl.BlockSpec((1, N_CLUSTERS, BINWIDTH),
                             lambda i, roi: (roi[i], 0, 0)),
            ],
            out_specs=pl.BlockSpec((1, N_CLUSTERS, BINWIDTH),
                                   lambda i, roi: (i, 0, 0)),
        ),
        out_shape=jax.ShapeDtypeStruct((N_ROI, N_CLUSTERS, BINWIDTH),
                                       jnp.float32),
    )(regions_oi, baseline_weight.reshape(n_regions, 1, BINWIDTH),
      delta_logit_weight)
    return table.reshape(-1)


def _emit_idx_lad(jb, in1_b, in2_b, idx_b, out_b, labels_v, g_v, lpi, cout,
                  iota16):
    sl = pl.ds(jb, 16)
    w1 = in1_b[sl]
    w2 = in2_b[sl]
    coord = w1 & 0x1FFFF
    fs = lax.shift_right_logical(w1, 17)
    cell = w2 & 0x3FFF
    region = lax.shift_right_logical(w2, 14)
    lw = plsc.load_gather(labels_v, [lax.shift_right_logical(cell, 3)])
    cl = lax.shift_right_logical(lw, lax.shift_left(cell & 7, 2)) & 7
    bin_ = lax.div(coord, BINSIZE)
    idx_b[sl] = (region * N_CLUSTERS + cl) * BINWIDTH + bin_
    fsf = fs.astype(jnp.float32)
    xb = jnp.clip(fsf * (1.0 / FS_WIDTH), 0.0, 1.0) * 4.0
    b = jnp.minimum(xb.astype(jnp.int32), 3)
    alpha = xb - b.astype(jnp.float32)
    gb = plsc.load_gather(g_v, [b])
    gb1 = plsc.load_gather(g_v, [b + 1])
    dens = gb + alpha * (gb1 - gb) + 1e-12
    ib = plsc.bitcast(dens, jnp.int32)
    e = (lax.shift_right_logical(ib, 23) & 0xFF) - 127
    m = plsc.bitcast((ib & 0x7FFFFF) | 0x3F800000, jnp.float32)
    p = jnp.float32(_LOG2_C[5])
    for k in (4, 3, 2, 1, 0):
        p = p * m + jnp.float32(_LOG2_C[k])
    ln = (e.astype(jnp.float32) + p) * _LN2
    lad = jnp.where(fsf > FS_WIDTH, cout, ln + lpi)
    plsc.store_scatter(out_b, [jb + iota16, iota16 * 0 + 1], lad)


def _sc_body(table_hbm, labels_hbm, w1_hbm, w2_hbm, consts_hbm, out_hbm,
             labels_v, g_v, in1_v0, in1_v1, in2_v0, in2_v1,
             idx_v0, idx_v1, gath_v0, gath_v1, out_v0, out_v1,
             s10, s11, s20, s21, so0, so1, sg0, sg1):
    in1_bufs = (in1_v0, in1_v1)
    in2_bufs = (in2_v0, in2_v1)
    out_bufs = (out_v0, out_v1)
    in1_sems = (s10, s11)
    in2_sems = (s20, s21)
    out_sems = (so0, so1)
    idx_bufs = (idx_v0, idx_v1)
    gath_bufs = (gath_v0, gath_v1)
    g_sems = (sg0, sg1)
    wid = lax.axis_index("s") * 2 + lax.axis_index("c")
    tile_base = wid * PER_TILE
    pltpu.sync_copy(labels_hbm, labels_v)
    pltpu.sync_copy(consts_hbm, g_v)
    iota16 = lax.iota(jnp.int32, 16)
    c5 = iota16 * 0 + 5
    lpi = plsc.load_gather(g_v, [c5])
    cout = plsc.load_gather(g_v, [c5 + 1])

    def in1_copy(g, b):
        base = tile_base + g * CHUNK
        return pltpu.make_async_copy(
            w1_hbm.at[pl.ds(base, CHUNK)], in1_bufs[b], in1_sems[b])

    def in2_copy(g, b):
        base = tile_base + g * CHUNK
        return pltpu.make_async_copy(
            w2_hbm.at[pl.ds(base, CHUNK)], in2_bufs[b], in2_sems[b])

    def in_start(g, b):
        in1_copy(g, b).start()
        in2_copy(g, b).start()

    def in_wait(g, b):
        in1_copy(g, b).wait()
        in2_copy(g, b).wait()

    def out_copy(g, b):
        base = tile_base + g * CHUNK
        return pltpu.make_async_copy(
            out_bufs[b], out_hbm.at[pl.ds(base, CHUNK)], out_sems[b])

    def scatter_lp0(out_b, jb, lp0):
        plsc.store_scatter(out_b, [jb + iota16, iota16 * 0], lp0)

    def gather_copy(b):
        return pltpu.make_async_copy(table_hbm.at[idx_bufs[b]], gath_bufs[b],
                                     g_sems[b])

    def pass1(b):
        def blk1(t, carry):
            for jj in range(9):
                _emit_idx_lad(t * 144 + jj * 16, in1_bufs[b], in2_bufs[b],
                              idx_bufs[b], out_bufs[b], labels_v, g_v, lpi,
                              cout, iota16)
            return carry

        lax.fori_loop(0, GROUPS // 9, blk1, 0)

    def pass2(b):
        def blk2(t, carry):
            for jj in range(9):
                jb = t * 144 + jj * 16
                scatter_lp0(out_bufs[b], jb, gath_bufs[b][pl.ds(jb, 16)])
            return carry

        lax.fori_loop(0, GROUPS // 9, blk2, 0)

    def compute(b):
        pass1(b)
        gather_copy(b).start()
        gather_copy(b).wait()
        pass2(b)

    in_start(0, 0)
    in_start(1, 1)
    in_wait(0, 0)
    pass1(0)
    gather_copy(0).start()

    def pipe_body(i, carry):
        for off, b in ((0, 1), (1, 0)):
            g = 1 + 2 * i + off

            @pl.when(g + 1 < NCHUNK)
            def _():
                in_start(g + 1, 1 - b)

            in_wait(g, b)

            @pl.when(g >= 2)
            def _():
                out_copy(g - 2, b).wait()

            pass1(b)
            gather_copy(b).start()
            gather_copy(1 - b).wait()
            pass2(1 - b)
            out_copy(g - 1, 1 - b).start()
        return carry

    lax.fori_loop(0, (NCHUNK - 1) // 2, pipe_body, 0)
    gather_copy(0).wait()
    pass2(0)
    out_copy(NCHUNK - 1, 0).start()
    out_copy(NCHUNK - 2, 1).wait()
    out_copy(NCHUNK - 1, 0).wait()

    @pl.when(wid == 0)
    def _tail():
        pltpu.sync_copy(w1_hbm.at[pl.ds(TAIL_BASE, TAIL)],
                        in1_v0.at[pl.ds(0, TAIL)])
        pltpu.sync_copy(w2_hbm.at[pl.ds(TAIL_BASE, TAIL)],
                        in2_v0.at[pl.ds(0, TAIL)])
        for j in range(TAIL // 16):
            _emit_idx_lad(j * 16, in1_v0, in2_v0, idx_v0, out_v0,
                          labels_v, g_v, lpi, cout, iota16)
        pltpu.async_copy(table_hbm.at[idx_v0.at[pl.ds(0, TAIL)]],
                         gath_v0.at[pl.ds(0, TAIL)], sg0).wait()
        for j in range(TAIL // 16):
            scatter_lp0(out_v0, j * 16, gath_v0[pl.ds(j * 16, 16)])
        pltpu.sync_copy(out_v0.at[pl.ds(0, TAIL)],
                        out_hbm.at[pl.ds(TAIL_BASE, TAIL)])


@functools.cache
def _sc_call():
    return pl.kernel(
        _sc_body,
        out_type=jax.ShapeDtypeStruct((N_FRAG, 2), jnp.float32),
        mesh=plsc.VectorSubcoreMesh(core_axis_name="c", subcore_axis_name="s",
                                    num_cores=2, num_subcores=16),
        compiler_params=pltpu.CompilerParams(needs_layout_passes=False,
                                             use_tc_tiling_on_sc=False),
        scratch_types=[
            pltpu.VMEM((LABEL_WORDS,), jnp.int32),
            pltpu.VMEM((128,), jnp.float32),
            pltpu.VMEM((CHUNK,), jnp.int32),
            pltpu.VMEM((CHUNK,), jnp.int32),
            pltpu.VMEM((CHUNK,), jnp.int32),
            pltpu.VMEM((CHUNK,), jnp.int32),
            pltpu.VMEM((CHUNK,), jnp.int32),
            pltpu.VMEM((CHUNK,), jnp.int32),
            pltpu.VMEM((CHUNK,), jnp.float32),
            pltpu.VMEM((CHUNK,), jnp.float32),
            pltpu.VMEM((CHUNK, 2), jnp.float32),
            pltpu.VMEM((CHUNK, 2), jnp.float32),
            pltpu.SemaphoreType.DMA,
            pltpu.SemaphoreType.DMA,
            pltpu.SemaphoreType.DMA,
            pltpu.SemaphoreType.DMA,
            pltpu.SemaphoreType.DMA,
            pltpu.SemaphoreType.DMA,
            pltpu.SemaphoreType.DMA,
            pltpu.SemaphoreType.DMA,
        ],
    )


def kernel(coord_left, fragment_size, regions_oi, local_region_ix,
           local_cell_ix, labels, baseline_weight, delta_logit_weight,
           spline_unnorm_heights, logprob_inside, spline_widths,
           spline_bin_locations):
    table = _build_table(regions_oi, baseline_weight, delta_logit_weight)
    lw = labels.astype(jnp.int32).reshape(LABEL_WORDS, 8)
    shifts = (jnp.arange(8, dtype=jnp.int32) * 4)[None, :]
    labels_words = jnp.sum(lw << shifts, axis=1, dtype=jnp.int32)
    w1 = coord_left.astype(jnp.int32) | (fragment_size.astype(jnp.int32) << 17)
    w2 = local_cell_ix.astype(jnp.int32) | (local_region_ix.astype(jnp.int32) << 14)
    h = jnp.exp(spline_unnorm_heights)
    norm = jnp.sum((h[:-1] + h[1:]) * 0.5 * spline_widths)
    g = (h / norm).astype(jnp.float32)
    lpi = logprob_inside.astype(jnp.float32)
    cout = jnp.log(1.0 - jnp.exp(lpi)) + lpi
    consts = jnp.concatenate(
        [g, lpi[None], cout[None], jnp.zeros((121,), jnp.float32)])
    return _sc_call()(table, labels_words, w1, w2, consts)


# speedup vs baseline: 1.7019x; 1.0015x over previous
import functools

import jax
import jax.numpy as jnp
from jax import lax
from jax.experimental import pallas as pl
from jax.experimental.pallas import tpu as pltpu
from jax.experimental.pallas import tpu_sc as plsc

BINSIZE = 200
FS_WIDTH = 1024.0
N_ROI = 64
N_CLUSTERS = 8
BINWIDTH = 500
TABLE_WORDS = N_ROI * N_CLUSTERS * BINWIDTH
N_FRAG = 1_000_000
NW = 32
CHUNK = 1488
GROUPS = CHUNK // 16
NCHUNK = 21
PER_TILE = CHUNK * NCHUNK
TAIL_BASE = PER_TILE * NW
TAIL = N_FRAG - TAIL_BASE
LABEL_WORDS = 1250

_LOG2_C = (-2.7868055642987652, 5.046852935527453, -3.4924660425540925,
           1.5938845482669501, -0.40486230941537504, 0.04342836333154978)
_LN2 = 0.6931471805599453


def _table_body(roi_ref, base_ref, delta_ref, out_ref):
    u = base_ref[0] + delta_ref[0]
    m = jnp.max(u, axis=-1, keepdims=True)
    lse = m + jnp.log(jnp.sum(jnp.exp(u - m), axis=-1, keepdims=True))
    out_ref[0] = u - lse


def _build_table(regions_oi, baseline_weight, delta_logit_weight):
    n_regions = baseline_weight.shape[0]
    table = pl.pallas_call(
        _table_body,
        grid_spec=pltpu.PrefetchScalarGridSpec(
            num_scalar_prefetch=1,
            grid=(N_ROI,),
            in_specs=[
                pl.BlockSpec((1, 1, BINWIDTH), lambda i, roi: (roi[i], 0, 0)),
                pl.BlockSpec((1, N_CLUSTERS, BINWIDTH),
                             lambda i, roi: (roi[i], 0, 0)),
            ],
            out_specs=pl.BlockSpec((1, N_CLUSTERS, BINWIDTH),
                                   lambda i, roi: (i, 0, 0)),
        ),
        out_shape=jax.ShapeDtypeStruct((N_ROI, N_CLUSTERS, BINWIDTH),
                                       jnp.float32),
    )(regions_oi, baseline_weight.reshape(n_regions, 1, BINWIDTH),
      delta_logit_weight)
    return table.reshape(-1)


def _emit_idx_lad(jb, in1_b, in2_b, idx_b, out_b, labels_v, g_v, lpi, cout,
                  iota16):
    sl = pl.ds(jb, 16)
    w1 = in1_b[sl]
    w2 = in2_b[sl]
    coord = w1 & 0x1FFFF
    fs = lax.shift_right_logical(w1, 17)
    cell = w2 & 0x3FFF
    region = lax.shift_right_logical(w2, 14)
    lw = plsc.load_gather(labels_v, [lax.shift_right_logical(cell, 3)])
    cl = lax.shift_right_logical(lw, lax.shift_left(cell & 7, 2)) & 7
    bin_ = lax.div(coord, BINSIZE)
    idx_b[sl] = (region * N_CLUSTERS + cl) * BINWIDTH + bin_
    fsf = fs.astype(jnp.float32)
    xb = jnp.clip(fsf * (1.0 / FS_WIDTH), 0.0, 1.0) * 4.0
    b = jnp.minimum(xb.astype(jnp.int32), 3)
    alpha = xb - b.astype(jnp.float32)
    gb = plsc.load_gather(g_v, [b])
    gb1 = plsc.load_gather(g_v, [b + 1])
    dens = gb + alpha * (gb1 - gb) + 1e-12
    ib = plsc.bitcast(dens, jnp.int32)
    e = (lax.shift_right_logical(ib, 23) & 0xFF) - 127
    m = plsc.bitcast((ib & 0x7FFFFF) | 0x3F800000, jnp.float32)
    p = jnp.float32(_LOG2_C[5])
    for k in (4, 3, 2, 1, 0):
        p = p * m + jnp.float32(_LOG2_C[k])
    ln = (e.astype(jnp.float32) + p) * _LN2
    lad = jnp.where(fsf > FS_WIDTH, cout, ln + lpi)
    plsc.store_scatter(out_b, [jb + iota16, iota16 * 0 + 1], lad)


def _sc_body(table_hbm, labels_hbm, w1_hbm, w2_hbm, consts_hbm, out_hbm,
             labels_v, g_v, in1_v0, in1_v1, in2_v0, in2_v1,
             idx_v0, idx_v1, gath_v0, gath_v1, out_v0, out_v1,
             s10, s11, s20, s21, so0, so1, sg0, sg1):
    in1_bufs = (in1_v0, in1_v1)
    in2_bufs = (in2_v0, in2_v1)
    out_bufs = (out_v0, out_v1)
    in1_sems = (s10, s11)
    in2_sems = (s20, s21)
    out_sems = (so0, so1)
    idx_bufs = (idx_v0, idx_v1)
    gath_bufs = (gath_v0, gath_v1)
    g_sems = (sg0, sg1)
    wid = lax.axis_index("s") * 2 + lax.axis_index("c")
    tile_base = wid * PER_TILE
    pltpu.sync_copy(labels_hbm, labels_v)
    pltpu.sync_copy(consts_hbm, g_v)
    iota16 = lax.iota(jnp.int32, 16)
    c5 = iota16 * 0 + 5
    lpi = plsc.load_gather(g_v, [c5])
    cout = plsc.load_gather(g_v, [c5 + 1])

    def in1_copy(g, b):
        base = tile_base + g * CHUNK
        return pltpu.make_async_copy(
            w1_hbm.at[pl.ds(base, CHUNK)], in1_bufs[b], in1_sems[b])

    def in2_copy(g, b):
        base = tile_base + g * CHUNK
        return pltpu.make_async_copy(
            w2_hbm.at[pl.ds(base, CHUNK)], in2_bufs[b], in2_sems[b])

    def in_start(g, b):
        in1_copy(g, b).start()
        in2_copy(g, b).start()

    def in_wait(g, b):
        in1_copy(g, b).wait()
        in2_copy(g, b).wait()

    def out_copy(g, b):
        base = tile_base + g * CHUNK
        return pltpu.make_async_copy(
            out_bufs[b], out_hbm.at[pl.ds(base, CHUNK)], out_sems[b])

    def scatter_lp0(out_b, jb, lp0):
        plsc.store_scatter(out_b, [jb + iota16, iota16 * 0], lp0)

    def gather_copy(b):
        return pltpu.make_async_copy(table_hbm.at[idx_bufs[b]], gath_bufs[b],
                                     g_sems[b])

    def pass1(b):
        def blk1(t, carry):
            for jj in range(3):
                _emit_idx_lad(t * 48 + jj * 16, in1_bufs[b], in2_bufs[b],
                              idx_bufs[b], out_bufs[b], labels_v, g_v, lpi,
                              cout, iota16)
            return carry

        lax.fori_loop(0, GROUPS // 3, blk1, 0)

    def pass2(b):
        def blk2(t, carry):
            for jj in range(3):
                jb = t * 48 + jj * 16
                scatter_lp0(out_bufs[b], jb, gath_bufs[b][pl.ds(jb, 16)])
            return carry

        lax.fori_loop(0, GROUPS // 3, blk2, 0)

    def compute(b):
        pass1(b)
        gather_copy(b).start()
        gather_copy(b).wait()
        pass2(b)

    in_start(0, 0)
    in_start(1, 1)
    in_wait(0, 0)
    pass1(0)
    gather_copy(0).start()

    def pipe_body(i, carry):
        for off, b in ((0, 1), (1, 0)):
            g = 1 + 2 * i + off

            @pl.when(g + 1 < NCHUNK)
            def _():
                in_start(g + 1, 1 - b)

            in_wait(g, b)

            @pl.when(g >= 2)
            def _():
                out_copy(g - 2, b).wait()

            pass1(b)
            gather_copy(b).start()
            gather_copy(1 - b).wait()
            pass2(1 - b)
            out_copy(g - 1, 1 - b).start()
        return carry

    lax.fori_loop(0, (NCHUNK - 1) // 2, pipe_body, 0)
    gather_copy(0).wait()
    pass2(0)
    out_copy(NCHUNK - 1, 0).start()
    out_copy(NCHUNK - 2, 1).wait()
    out_copy(NCHUNK - 1, 0).wait()

    @pl.when(wid == 0)
    def _tail():
        pltpu.sync_copy(w1_hbm.at[pl.ds(TAIL_BASE, TAIL)],
                        in1_v0.at[pl.ds(0, TAIL)])
        pltpu.sync_copy(w2_hbm.at[pl.ds(TAIL_BASE, TAIL)],
                        in2_v0.at[pl.ds(0, TAIL)])
        for j in range(TAIL // 16):
            _emit_idx_lad(j * 16, in1_v0, in2_v0, idx_v0, out_v0,
                          labels_v, g_v, lpi, cout, iota16)
        pltpu.async_copy(table_hbm.at[idx_v0.at[pl.ds(0, TAIL)]],
                         gath_v0.at[pl.ds(0, TAIL)], sg0).wait()
        for j in range(TAIL // 16):
            scatter_lp0(out_v0, j * 16, gath_v0[pl.ds(j * 16, 16)])
        pltpu.sync_copy(out_v0.at[pl.ds(0, TAIL)],
                        out_hbm.at[pl.ds(TAIL_BASE, TAIL)])


@functools.cache
def _sc_call():
    return pl.kernel(
        _sc_body,
        out_type=jax.ShapeDtypeStruct((N_FRAG, 2), jnp.float32),
        mesh=plsc.VectorSubcoreMesh(core_axis_name="c", subcore_axis_name="s",
                                    num_cores=2, num_subcores=16),
        compiler_params=pltpu.CompilerParams(needs_layout_passes=False,
                                             use_tc_tiling_on_sc=False),
        scratch_types=[
            pltpu.VMEM((LABEL_WORDS,), jnp.int32),
            pltpu.VMEM((128,), jnp.float32),
            pltpu.VMEM((CHUNK,), jnp.int32),
            pltpu.VMEM((CHUNK,), jnp.int32),
            pltpu.VMEM((CHUNK,), jnp.int32),
            pltpu.VMEM((CHUNK,), jnp.int32),
            pltpu.VMEM((CHUNK,), jnp.int32),
            pltpu.VMEM((CHUNK,), jnp.int32),
            pltpu.VMEM((CHUNK,), jnp.float32),
            pltpu.VMEM((CHUNK,), jnp.float32),
            pltpu.VMEM((CHUNK, 2), jnp.float32),
            pltpu.VMEM((CHUNK, 2), jnp.float32),
            pltpu.SemaphoreType.DMA,
            pltpu.SemaphoreType.DMA,
            pltpu.SemaphoreType.DMA,
            pltpu.SemaphoreType.DMA,
            pltpu.SemaphoreType.DMA,
            pltpu.SemaphoreType.DMA,
            pltpu.SemaphoreType.DMA,
            pltpu.SemaphoreType.DMA,
        ],
    )


def kernel(coord_left, fragment_size, regions_oi, local_region_ix,
           local_cell_ix, labels, baseline_weight, delta_logit_weight,
           spline_unnorm_heights, logprob_inside, spline_widths,
           spline_bin_locations):
    table = _build_table(regions_oi, baseline_weight, delta_logit_weight)
    lw = labels.astype(jnp.int32).reshape(LABEL_WORDS, 8)
    shifts = (jnp.arange(8, dtype=jnp.int32) * 4)[None, :]
    labels_words = jnp.sum(lw << shifts, axis=1, dtype=jnp.int32)
    w1 = coord_left.astype(jnp.int32) | (fragment_size.astype(jnp.int32) << 17)
    w2 = local_cell_ix.astype(jnp.int32) | (local_region_ix.astype(jnp.int32) << 14)
    h = jnp.exp(spline_unnorm_heights)
    norm = jnp.sum((h[:-1] + h[1:]) * 0.5 * spline_widths)
    g = (h / norm).astype(jnp.float32)
    lpi = logprob_inside.astype(jnp.float32)
    cout = jnp.log(1.0 - jnp.exp(lpi)) + lpi
    consts = jnp.concatenate(
        [g, lpi[None], cout[None], jnp.zeros((121,), jnp.float32)])
    return _sc_call()(table, labels_words, w1, w2, consts)
